# baseline (speedup 1.0000x reference)
import functools

import jax
import jax.numpy as jnp
from jax.experimental import pallas as pl
from jax.experimental.pallas import tpu as pltpu

N = 4096
DIN = 128
DH = 64
DOUT = 128
H = 4

LOG2E = 1.4426950408889634
NEG = -9e15 * LOG2E


def _proj_body(x_ref, w_ref, am_ref, h_ref, f_ref):
    h = jnp.dot(x_ref[...], w_ref[...], preferred_element_type=jnp.float32)
    h_ref[...] = h
    f_ref[...] = jnp.dot(h, am_ref[...], preferred_element_type=jnp.float32)


def _proj(x, w, am, bi):
    n, din = x.shape
    dh_tot = w.shape[1]
    return pl.pallas_call(
        _proj_body,
        grid=(n // bi,),
        in_specs=[
            pl.BlockSpec((bi, din), lambda i: (i, 0)),
            pl.BlockSpec((din, dh_tot), lambda i: (0, 0)),
            pl.BlockSpec((dh_tot, 8), lambda i: (0, 0)),
        ],
        out_specs=[
            pl.BlockSpec((bi, dh_tot), lambda i: (i, 0)),
            pl.BlockSpec((bi, 8), lambda i: (i, 0)),
        ],
        out_shape=[
            jax.ShapeDtypeStruct((n, dh_tot), jnp.float32),
            jax.ShapeDtypeStruct((n, 8), jnp.float32),
        ],
        compiler_params=pltpu.CompilerParams(
            dimension_semantics=("parallel",),
        ),
    )(x, w, am)


def _flash_body(nheads, dh, adj_ref, a_ref, h_ref, fb_ref, ft_ref, *rest):
    adj = adj_ref[...]
    ab = a_ref[...]
    outs = []
    for hh in range(nheads):
        f1c = fb_ref[:, hh : hh + 1]
        f2r = ft_ref[nheads + hh : nheads + hh + 1, :]
        e = f1c + f2r
        e = jnp.maximum(e, 0.2 * e)
        e = jnp.where(adj > 0.0, e, NEG)
        m = jnp.max(e, axis=1, keepdims=True)
        p = jnp.exp2(e - m)
        z = jnp.sum(p, axis=1, keepdims=True)
        o = jnp.dot(
            (p * ab).astype(jnp.bfloat16),
            h_ref[:, hh * dh : (hh + 1) * dh].astype(jnp.bfloat16),
            preferred_element_type=jnp.float32,
        )
        o = o / z
        o = jnp.where(o > 0.0, o, jnp.exp(jnp.minimum(o, 0.0)) - 1.0)
        outs.append(o)
    xm = outs[0] if nheads == 1 else jnp.concatenate(outs, axis=1)
    if len(rest) == 3:
        w2_ref, a2m_ref, h2_ref, f2_ref = rest[0], rest[1], rest[2][0], rest[2][1]
        h2 = jnp.dot(xm, w2_ref[...], preferred_element_type=jnp.float32)
        h2_ref[...] = h2
        f2_ref[...] = jnp.dot(h2, a2m_ref[...], preferred_element_type=jnp.float32)
    else:
        rest[0][...] = xm


def _flash(adj, a, h, fb, ft, nheads, dh, bi, w2=None, a2m=None):
    n = adj.shape[0]
    dh_tot = nheads * dh
    in_specs = [
        pl.BlockSpec((bi, n), lambda i: (i, 0)),
        pl.BlockSpec((bi, n), lambda i: (i, 0)),
        pl.BlockSpec((n, dh_tot), lambda i: (0, 0)),
        pl.BlockSpec((bi, 8), lambda i: (i, 0)),
        pl.BlockSpec((8, n), lambda i: (0, 0)),
    ]
    args = [adj, a, h, fb, ft]
    if w2 is not None:
        dout = w2.shape[1]
        in_specs += [
            pl.BlockSpec((dh_tot, dout), lambda i: (0, 0)),
            pl.BlockSpec((dout, 8), lambda i: (0, 0)),
        ]
        args += [w2, a2m]
        out_specs = [
            pl.BlockSpec((bi, dout), lambda i: (i, 0)),
            pl.BlockSpec((bi, 8), lambda i: (i, 0)),
        ]
        out_shape = [
            jax.ShapeDtypeStruct((n, dout), jnp.float32),
            jax.ShapeDtypeStruct((n, 8), jnp.float32),
        ]

        def body(adj_ref, a_ref, h_ref, fb_ref, ft_ref, w2_ref, a2m_ref, o1, o2):
            _flash_body(nheads, dh, adj_ref, a_ref, h_ref, fb_ref, ft_ref,
                        w2_ref, a2m_ref, (o1, o2))
    else:
        out_specs = [pl.BlockSpec((bi, dh_tot), lambda i: (i, 0))]
        out_shape = [jax.ShapeDtypeStruct((n, dh_tot), jnp.float32)]

        def body(adj_ref, a_ref, h_ref, fb_ref, ft_ref, o1):
            _flash_body(nheads, dh, adj_ref, a_ref, h_ref, fb_ref, ft_ref, o1)

    return pl.pallas_call(
        body,
        grid=(n // bi,),
        in_specs=in_specs,
        out_specs=out_specs,
        out_shape=out_shape,
        compiler_params=pltpu.CompilerParams(
            dimension_semantics=("parallel",),
        ),
    )(*args)


@jax.jit
def kernel(x, adj, A, W1, a1, W2, a2):
    w1r = jnp.transpose(W1, (1, 0, 2)).reshape(DIN, H * DH)
    eye = jnp.eye(H, dtype=jnp.float32)
    m1 = (eye[:, None, :] * a1[:, :DH][:, :, None]).reshape(H * DH, H)
    m2 = (eye[:, None, :] * a1[:, DH:][:, :, None]).reshape(H * DH, H)
    a1m = jnp.concatenate([m1, m2], axis=1) * LOG2E
    a2m = jnp.zeros((DOUT, 8), jnp.float32)
    a2m = a2m.at[:, 0].set(a2[:DOUT]).at[:, 1].set(a2[DOUT:]) * LOG2E

    h1, f1 = _proj(x, w1r, a1m, 512)
    h2, g = _flash(adj, A, h1, f1, f1.T, H, DH, 256, w2=W2, a2m=a2m)
    out = _flash(adj, A, h2, g, g.T, 1, DOUT, 256)
    return out[0]


# speedup vs baseline: 1.0784x; 1.0784x over previous
import functools

import jax
import jax.numpy as jnp
from jax.experimental import pallas as pl
from jax.experimental.pallas import tpu as pltpu

N = 4096
DIN = 128
DH = 64
DOUT = 128
H = 4

LOG2E = 1.4426950408889634
NEG = -9e15 * LOG2E


def _proj_body(x_ref, w_ref, am_ref, h_ref, f_ref):
    h = jnp.dot(x_ref[...], w_ref[...], preferred_element_type=jnp.float32)
    h_ref[...] = h
    f_ref[...] = jnp.dot(h, am_ref[...], preferred_element_type=jnp.float32)


def _proj(x, w, am, bi):
    n, din = x.shape
    dh_tot = w.shape[1]
    return pl.pallas_call(
        _proj_body,
        grid=(n // bi,),
        in_specs=[
            pl.BlockSpec((bi, din), lambda i: (i, 0)),
            pl.BlockSpec((din, dh_tot), lambda i: (0, 0)),
            pl.BlockSpec((dh_tot, 8), lambda i: (0, 0)),
        ],
        out_specs=[
            pl.BlockSpec((bi, dh_tot), lambda i: (i, 0)),
            pl.BlockSpec((bi, 8), lambda i: (i, 0)),
        ],
        out_shape=[
            jax.ShapeDtypeStruct((n, dh_tot), jnp.float32),
            jax.ShapeDtypeStruct((n, 8), jnp.float32),
        ],
        compiler_params=pltpu.CompilerParams(
            dimension_semantics=("parallel",),
        ),
    )(x, w, am)


def _flash_body(nheads, dh, adj_ref, a_ref, h_ref, fb_ref, ft_ref, *rest):
    adj = adj_ref[...]
    ab = a_ref[...]
    outs = []
    for hh in range(nheads):
        f1c = fb_ref[:, hh : hh + 1]
        f2r = ft_ref[nheads + hh : nheads + hh + 1, :]
        e = f1c + f2r
        e = jnp.maximum(e, 0.2 * e)
        e = jnp.where(adj > 0.0, e, NEG)
        m = jnp.max(e, axis=1, keepdims=True)
        p = jnp.exp2(e - m)
        z = jnp.sum(p, axis=1, keepdims=True)
        o = jnp.dot(
            p * ab,
            h_ref[:, hh * dh : (hh + 1) * dh],
            preferred_element_type=jnp.float32,
        )
        o = o / z
        o = jnp.where(o > 0.0, o, jnp.exp(jnp.minimum(o, 0.0)) - 1.0)
        outs.append(o)
    xm = outs[0] if nheads == 1 else jnp.concatenate(outs, axis=1)
    if len(rest) == 3:
        w2_ref, a2m_ref, h2_ref, f2_ref = rest[0], rest[1], rest[2][0], rest[2][1]
        h2 = jnp.dot(xm, w2_ref[...], preferred_element_type=jnp.float32)
        h2_ref[...] = h2
        f2_ref[...] = jnp.dot(h2, a2m_ref[...], preferred_element_type=jnp.float32)
    else:
        rest[0][...] = xm


def _flash(adj, a, h, fb, ft, nheads, dh, bi, w2=None, a2m=None):
    n = adj.shape[0]
    dh_tot = nheads * dh
    in_specs = [
        pl.BlockSpec((bi, n), lambda i: (i, 0)),
        pl.BlockSpec((bi, n), lambda i: (i, 0)),
        pl.BlockSpec((n, dh_tot), lambda i: (0, 0)),
        pl.BlockSpec((bi, 8), lambda i: (i, 0)),
        pl.BlockSpec((8, n), lambda i: (0, 0)),
    ]
    args = [adj, a, h, fb, ft]
    if w2 is not None:
        dout = w2.shape[1]
        in_specs += [
            pl.BlockSpec((dh_tot, dout), lambda i: (0, 0)),
            pl.BlockSpec((dout, 8), lambda i: (0, 0)),
        ]
        args += [w2, a2m]
        out_specs = [
            pl.BlockSpec((bi, dout), lambda i: (i, 0)),
            pl.BlockSpec((bi, 8), lambda i: (i, 0)),
        ]
        out_shape = [
            jax.ShapeDtypeStruct((n, dout), jnp.float32),
            jax.ShapeDtypeStruct((n, 8), jnp.float32),
        ]

        def body(adj_ref, a_ref, h_ref, fb_ref, ft_ref, w2_ref, a2m_ref, o1, o2):
            _flash_body(nheads, dh, adj_ref, a_ref, h_ref, fb_ref, ft_ref,
                        w2_ref, a2m_ref, (o1, o2))
    else:
        out_specs = [pl.BlockSpec((bi, dh_tot), lambda i: (i, 0))]
        out_shape = [jax.ShapeDtypeStruct((n, dh_tot), jnp.float32)]

        def body(adj_ref, a_ref, h_ref, fb_ref, ft_ref, o1):
            _flash_body(nheads, dh, adj_ref, a_ref, h_ref, fb_ref, ft_ref, o1)

    return pl.pallas_call(
        body,
        grid=(n // bi,),
        in_specs=in_specs,
        out_specs=out_specs,
        out_shape=out_shape,
        compiler_params=pltpu.CompilerParams(
            dimension_semantics=("parallel",),
        ),
    )(*args)


@jax.jit
def kernel(x, adj, A, W1, a1, W2, a2):
    w1r = jnp.transpose(W1, (1, 0, 2)).reshape(DIN, H * DH)
    eye = jnp.eye(H, dtype=jnp.float32)
    m1 = (eye[:, None, :] * a1[:, :DH][:, :, None]).reshape(H * DH, H)
    m2 = (eye[:, None, :] * a1[:, DH:][:, :, None]).reshape(H * DH, H)
    a1m = jnp.concatenate([m1, m2], axis=1) * LOG2E
    a2m = jnp.zeros((DOUT, 8), jnp.float32)
    a2m = a2m.at[:, 0].set(a2[:DOUT]).at[:, 1].set(a2[DOUT:]) * LOG2E

    h1, f1 = _proj(x, w1r, a1m, 512)
    h2, g = _flash(adj, A, h1, f1, f1.T, H, DH, 256, w2=W2, a2m=a2m)
    out = _flash(adj, A, h2, g, g.T, 1, DOUT, 256)
    return out[0]
